# Initial kernel scaffold; baseline (speedup 1.0000x reference)
import jax
import jax.numpy as jnp
from jax.experimental import pallas as pl


def kernel(encodings, subnetwork, W, att_src, att_dst, bias):
    raise NotImplementedError("write your pallas kernel here")


# speedup vs baseline: 38.6701x; 38.6701x over previous
import functools

import jax
import jax.numpy as jnp
from jax import lax
from jax.experimental import pallas as pl
from jax.experimental.pallas import tpu as pltpu
from jax.experimental.pallas import tpu_sc as plsc

NEG = 0.2
NC, NS, LANES = 2, 16, 16
NW = NC * NS


def _build_tables(enc, W, a_src_pad, a_dst_pad, srow):
    n, d = enc.shape
    hc = W.shape[1]
    bs = 2000 if n % 2000 == 0 else n

    def body(enc_ref, w_ref, as_ref, ad_ref, s_ref, t_ref):
        xw = jnp.dot(enc_ref[...], w_ref[...], preferred_element_type=jnp.float32)
        s_ref[:, 0:hc] = xw
        s_ref[:, hc:srow] = jnp.dot(xw, as_ref[...], preferred_element_type=jnp.float32)
        t_ref[...] = jnp.dot(xw, ad_ref[...], preferred_element_type=jnp.float32)

    return pl.pallas_call(
        body,
        grid=(n // bs,),
        in_specs=[
            pl.BlockSpec((bs, d), lambda i: (i, 0)),
            pl.BlockSpec((d, hc), lambda i: (0, 0)),
            pl.BlockSpec((d, 16), lambda i: (0, 0)),
            pl.BlockSpec((d, 16), lambda i: (0, 0)),
        ],
        out_specs=[
            pl.BlockSpec((bs, srow), lambda i: (i, 0)),
            pl.BlockSpec((bs, 16), lambda i: (i, 0)),
        ],
        out_shape=[
            jax.ShapeDtypeStruct((n, srow), jnp.float32),
            jax.ShapeDtypeStruct((n, 16), jnp.float32),
        ],
    )(enc, W, a_src_pad, a_dst_pad)


def _sc_edge_pass(S, T, e_src, e_dst, n, srow, h, c):
    e_total = e_src.shape[0]
    ept = e_total // NW
    ch = 80
    nch = ept // ch
    rpt = n // NS
    zr = 125
    hc = h * c
    mesh = plsc.VectorSubcoreMesh(
        core_axis_name="c", subcore_axis_name="s", num_cores=NC, num_subcores=NS
    )

    @functools.partial(
        pl.kernel,
        out_type=jax.ShapeDtypeStruct((NC * n, srow), jnp.float32),
        mesh=mesh,
        compiler_params=pltpu.CompilerParams(
            use_tc_tiling_on_sc=False, needs_layout_passes=False
        ),
        scratch_types=[
            pltpu.VMEM((ch,), jnp.int32),
            pltpu.VMEM((ch,), jnp.int32),
            pltpu.VMEM((ch, srow), jnp.float32),
            pltpu.VMEM((ch, 16), jnp.float32),
            pltpu.VMEM((zr, srow), jnp.float32),
            pltpu.VMEM_SHARED((n, srow), jnp.float32),
            pltpu.SemaphoreType.DMA,
            pltpu.SemaphoreType.DMA,
        ],
    )
    def body(s_hbm, t_hbm, src_hbm, dst_hbm, out_hbm, srcv, dstv, rowsv, adstv,
             zbuf, acc, sem1, sem2):
        cid = lax.axis_index("c")
        sid = lax.axis_index("s")
        wid = cid * NS + sid

        vper = srow // LANES

        def zb(k, carry):
            r = k // vper
            col = (k % vper) * LANES
            zbuf[r, pl.ds(col, LANES)] = jnp.zeros((LANES,), jnp.float32)
            return carry

        lax.fori_loop(0, zr * vper, zb, 0)

        def zc(q, carry):
            pltpu.sync_copy(zbuf, acc.at[pl.ds(sid * rpt + q * zr, zr)])
            return carry

        lax.fori_loop(0, rpt // zr, zc, 0)
        plsc.subcore_barrier()

        ebase = wid * ept

        def chunk(j, carry):
            base = ebase + j * ch
            pltpu.sync_copy(src_hbm.at[pl.ds(base, ch)], srcv)
            pltpu.sync_copy(dst_hbm.at[pl.ds(base, ch)], dstv)
            cp1 = pltpu.async_copy(s_hbm.at[srcv], rowsv, sem1)
            cp2 = pltpu.async_copy(t_hbm.at[dstv], adstv, sem2)
            cp1.wait()
            cp2.wait()

            def edge(e, ecarry):
                arow = rowsv[e, pl.ds(hc, LANES)]
                ad = adstv[e, :]
                al = arow + ad
                al = jnp.where(al >= 0, al, al * NEG)
                w = jnp.exp(al)
                rowsv[e, pl.ds(hc, LANES)] = w
                eidx = jnp.full((LANES,), e, jnp.int32)
                for hh in range(h):
                    coeff = plsc.load_gather(
                        rowsv, [eidx, jnp.full((LANES,), hc + hh, jnp.int32)]
                    )
                    for half in range(c // LANES):
                        off = hh * c + half * LANES
                        rowsv[e, pl.ds(off, LANES)] = (
                            rowsv[e, pl.ds(off, LANES)] * coeff
                        )
                return ecarry

            lax.fori_loop(0, ch, edge, 0)
            pltpu.sync_copy(rowsv, acc.at[dstv], add=True)
            return carry

        lax.fori_loop(0, nch, chunk, 0)
        plsc.subcore_barrier()

        def rb(q, carry):
            roff = sid * rpt + q * zr
            pltpu.sync_copy(acc.at[pl.ds(roff, zr)], zbuf)
            pltpu.sync_copy(zbuf, out_hbm.at[pl.ds(cid * n + roff, zr)])
            return carry

        lax.fori_loop(0, rpt // zr, rb, 0)

    return body(S, T, e_src, e_dst)


def _finalize(S, T, acc, rep, bias2d, n, srow, h, c):
    hc = h * c
    bs = 2000 if n % 2000 == 0 else n

    def body(s_ref, t_ref, acc_ref, r_ref, b_ref, o_ref):
        x = s_ref[:, 0:hc]
        asrc = s_ref[:, hc:hc + h]
        adst = t_ref[:, 0:h]
        al = asrc + adst
        al = jnp.where(al >= 0, al, al * NEG)
        wl = jnp.exp(al)
        accs = acc_ref[0] + acc_ref[1]
        numer = accs[:, 0:hc] + x * jnp.dot(
            wl, r_ref[...], preferred_element_type=jnp.float32
        )
        den = accs[:, hc:hc + h] + wl
        denr = jnp.dot(den, r_ref[...], preferred_element_type=jnp.float32)
        o_ref[...] = numer / (denr + 1e-16) + b_ref[...]

    return pl.pallas_call(
        body,
        grid=(n // bs,),
        in_specs=[
            pl.BlockSpec((bs, srow), lambda i: (i, 0)),
            pl.BlockSpec((bs, 16), lambda i: (i, 0)),
            pl.BlockSpec((NC, bs, srow), lambda i: (0, i, 0)),
            pl.BlockSpec((h, hc), lambda i: (0, 0)),
            pl.BlockSpec((1, hc), lambda i: (0, 0)),
        ],
        out_specs=pl.BlockSpec((bs, hc), lambda i: (i, 0)),
        out_shape=jax.ShapeDtypeStruct((n, hc), jnp.float32),
    )(S, T, acc, rep, bias2d)


def kernel(encodings, subnetwork, W, att_src, att_dst, bias):
    n, d = encodings.shape
    h, c = att_src.shape
    hc = h * c
    srow = hc + 16

    eye = jnp.eye(h, 16, dtype=jnp.float32)
    a_src_pad = (att_src[:, :, None] * eye[:, None, :]).reshape(hc, 16)
    a_dst_pad = (att_dst[:, :, None] * eye[:, None, :]).reshape(hc, 16)
    rep = jnp.repeat(jnp.eye(h, dtype=jnp.float32), c, axis=1)
    bias2d = bias.reshape(1, hc)

    S, T = _build_tables(encodings, W, a_src_pad, a_dst_pad, srow)
    acc = _sc_edge_pass(S, T, subnetwork[0], subnetwork[1], n, srow, h, c)
    acc = acc.reshape(NC, n, srow)
    return _finalize(S, T, acc, rep, bias2d, n, srow, h, c)
